# baseline (device time: 41350 ns/iter reference)
import jax
import jax.numpy as jnp
from jax import lax
from jax.experimental import pallas as pl
from jax.experimental.pallas import tpu as pltpu

N_DEV = 4
R = [0, 2, 3, 0]


def kernel(x, Win0, Wout0, Win1, Wout1, Win2, Wout2):
    m, d = x.shape

    def body(x_ref, win0_ref, wout0_ref, win1_ref, wout1_ref,
             win2_ref, wout2_ref, out_ref,
             xbufA_ref, xbufB_ref, part_ref, rcv_ref,
             winv_ref, woutv_ref,
             ag_send, ag_recv, rs_send, rs_recv, w_sem):
        my = lax.axis_index("i")
        left = lax.rem(my + 3, N_DEV)
        right = lax.rem(my + 1, N_DEV)
        diag = lax.rem(my + 2, N_DEV)

        def cix(off):
            return lax.rem(my + (off % N_DEV), N_DEV)

        def copy(src, dst, ssem, rsem, dev):
            return pltpu.make_async_remote_copy(
                src_ref=src, dst_ref=dst, send_sem=ssem, recv_sem=rsem,
                device_id=(dev,), device_id_type=pl.DeviceIdType.MESH)

        w_copies = []
        for i, (hsrc, vdst) in enumerate([
                (win0_ref, winv_ref.at[0]), (wout0_ref, woutv_ref.at[0]),
                (win1_ref, winv_ref.at[1]), (wout1_ref, woutv_ref.at[1]),
                (win2_ref, winv_ref.at[2]), (wout2_ref, woutv_ref.at[2])]):
            c = pltpu.make_async_copy(hsrc, vdst, w_sem.at[i])
            c.start()
            w_copies.append(c)

        barrier = pltpu.get_barrier_semaphore()
        for nbr in (left, right, diag):
            pl.semaphore_signal(barrier, inc=1, device_id=(nbr,),
                                device_id_type=pl.DeviceIdType.MESH)
        pl.semaphore_wait(barrier, 3)

        xbufA_ref[my] = x_ref[...]

        rs_sends = []

        for layer in range(3):
            w_copies[2 * layer].wait()
            w_copies[2 * layer + 1].wait()
            win = winv_ref.at[layer]
            wout = woutv_ref.at[layer]
            xbuf = xbufA_ref if layer % 2 == 0 else xbufB_ref
            xnext = xbufB_ref if layer % 2 == 0 else xbufA_ref
            r, rp = R[layer], R[layer + 1]
            delta = (rp - r) % N_DEV
            c_own = cix(-r)
            c_l = cix(-1 - r)
            c_r = cix(1 - r)
            c_d = cix(2 - r)
            c_keep = cix(-rp)

            def compute_chunk(c):
                hc = jnp.maximum(
                    jnp.dot(xbuf[c], win[...],
                            preferred_element_type=jnp.float32), 0.0)
                part_ref[c] = jnp.dot(
                    hc, wout[...], preferred_element_type=jnp.float32)

            ag_to_r = copy(xbuf.at[c_own], xbuf.at[c_own],
                           ag_send.at[0], ag_recv.at[0], right)
            ag_to_l = copy(xbuf.at[c_own], xbuf.at[c_own],
                           ag_send.at[1], ag_recv.at[1], left)
            ag_to_d = copy(xbuf.at[c_own], xbuf.at[c_own],
                           ag_send.at[2], ag_recv.at[2], diag)
            ag_to_r.start()
            ag_to_l.start()
            ag_to_d.start()
            for s in rs_sends:
                s.wait_send()

            ag_fr_l = copy(xbuf.at[c_l], xbuf.at[c_l],
                           ag_send.at[0], ag_recv.at[0], left)
            ag_fr_r = copy(xbuf.at[c_r], xbuf.at[c_r],
                           ag_send.at[1], ag_recv.at[1], right)
            ag_fr_d = copy(xbuf.at[c_d], xbuf.at[c_d],
                           ag_send.at[2], ag_recv.at[2], diag)

            def rs_to(dev, slot, c):
                s = copy(part_ref.at[c], rcv_ref.at[slot],
                         rs_send.at[slot], rs_recv.at[slot], dev)
                s.start()
                return s

            if delta == 2:
                compute_chunk(c_own)
                s_d = rs_to(diag, 2, c_own)
                ag_fr_l.wait_recv()
                compute_chunk(c_l)
                s_r = rs_to(right, 0, c_l)
                ag_fr_r.wait_recv()
                compute_chunk(c_r)
                s_l = rs_to(left, 1, c_r)
                ag_fr_d.wait_recv()
                compute_chunk(c_d)
            else:
                compute_chunk(c_own)
                s_r = rs_to(right, 0, c_own)
                ag_fr_r.wait_recv()
                compute_chunk(c_r)
                s_d = rs_to(diag, 2, c_r)
                ag_fr_l.wait_recv()
                compute_chunk(c_l)
                ag_fr_d.wait_recv()
                compute_chunk(c_d)
                s_l = rs_to(left, 1, c_d)
            rs_sends = [s_l, s_r, s_d]

            ag_to_r.wait_send()
            ag_to_l.wait_send()
            ag_to_d.wait_send()

            rs_fr_l = copy(part_ref.at[c_keep], rcv_ref.at[0],
                           rs_send.at[0], rs_recv.at[0], left)
            rs_fr_r = copy(part_ref.at[c_keep], rcv_ref.at[1],
                           rs_send.at[1], rs_recv.at[1], right)
            rs_fr_d = copy(part_ref.at[c_keep], rcv_ref.at[2],
                           rs_send.at[2], rs_recv.at[2], diag)
            rs_fr_l.wait_recv()
            rs_fr_r.wait_recv()
            rs_fr_d.wait_recv()
            y = (part_ref[c_keep] + rcv_ref[0]) + (rcv_ref[1] + rcv_ref[2])
            if layer < 2:
                xnext[c_keep] = y
            else:
                out_ref[...] = y
                for s in rs_sends:
                    s.wait_send()

    return pl.pallas_call(
        body,
        out_shape=jax.ShapeDtypeStruct((m, d), jnp.float32),
        in_specs=[pl.BlockSpec(memory_space=pltpu.VMEM)]
        + [pl.BlockSpec(memory_space=pl.ANY)] * 6,
        out_specs=pl.BlockSpec(memory_space=pltpu.VMEM),
        scratch_shapes=[
            pltpu.VMEM((N_DEV, m, d), jnp.float32),
            pltpu.VMEM((N_DEV, m, d), jnp.float32),
            pltpu.VMEM((N_DEV, m, d), jnp.float32),
            pltpu.VMEM((N_DEV - 1, m, d), jnp.float32),
            pltpu.VMEM((3,) + Win0.shape, jnp.float32),
            pltpu.VMEM((3,) + Wout0.shape, jnp.float32),
            pltpu.SemaphoreType.DMA((N_DEV - 1,)),
            pltpu.SemaphoreType.DMA((N_DEV - 1,)),
            pltpu.SemaphoreType.DMA((N_DEV - 1,)),
            pltpu.SemaphoreType.DMA((N_DEV - 1,)),
            pltpu.SemaphoreType.DMA((6,)),
        ],
        compiler_params=pltpu.CompilerParams(collective_id=0),
    )(x, Win0, Wout0, Win1, Wout1, Win2, Wout2)


# device time: 41115 ns/iter; 1.0057x vs baseline; 1.0057x over previous
import jax
import jax.numpy as jnp
from jax import lax
from jax.experimental import pallas as pl
from jax.experimental.pallas import tpu as pltpu

N_DEV = 4
R = [0, 2, 3, 0]


def kernel(x, Win0, Wout0, Win1, Wout1, Win2, Wout2):
    m, d = x.shape

    def body(x_ref, win0_ref, wout0_ref, win1_ref, wout1_ref,
             win2_ref, wout2_ref, out_ref,
             xbufA_ref, xbufB_ref, part_ref, rcv_ref,
             winv_ref, woutv_ref,
             ag_send, ag_recv, rs_send, rs_recv, w_sem):
        my = lax.axis_index("i")
        left = lax.rem(my + 3, N_DEV)
        right = lax.rem(my + 1, N_DEV)
        diag = lax.rem(my + 2, N_DEV)

        def cix(off):
            return lax.rem(my + (off % N_DEV), N_DEV)

        def copy(src, dst, ssem, rsem, dev):
            return pltpu.make_async_remote_copy(
                src_ref=src, dst_ref=dst, send_sem=ssem, recv_sem=rsem,
                device_id=(dev,), device_id_type=pl.DeviceIdType.MESH)

        w_copies = []
        for i, (hsrc, vdst) in enumerate([
                (win0_ref, winv_ref.at[0]), (wout0_ref, woutv_ref.at[0]),
                (win1_ref, winv_ref.at[1]), (wout1_ref, woutv_ref.at[1]),
                (win2_ref, winv_ref.at[2]), (wout2_ref, woutv_ref.at[2])]):
            c = pltpu.make_async_copy(hsrc, vdst, w_sem.at[i])
            c.start()
            w_copies.append(c)

        barrier = pltpu.get_barrier_semaphore()
        for nbr in (left, right, diag):
            pl.semaphore_signal(barrier, inc=1, device_id=(nbr,),
                                device_id_type=pl.DeviceIdType.MESH)
        pl.semaphore_wait(barrier, 3)

        xbufA_ref[my] = x_ref[...]

        rs_sends = []

        for layer in range(3):
            w_copies[2 * layer].wait()
            w_copies[2 * layer + 1].wait()
            win = winv_ref.at[layer]
            wout = woutv_ref.at[layer]
            xbuf = xbufA_ref if layer % 2 == 0 else xbufB_ref
            xnext = xbufB_ref if layer % 2 == 0 else xbufA_ref
            r, rp = R[layer], R[layer + 1]
            delta = (rp - r) % N_DEV
            c_own = cix(-r)
            c_l = cix(-1 - r)
            c_r = cix(1 - r)
            c_d = cix(2 - r)
            c_keep = cix(-rp)

            def compute_chunk(c):
                hc = jnp.maximum(
                    jnp.dot(xbuf[c], win[...],
                            preferred_element_type=jnp.float32), 0.0)
                part_ref[c] = jnp.dot(
                    hc, wout[...], preferred_element_type=jnp.float32)

            ag_to_r = copy(xbuf.at[c_own], xbuf.at[c_own],
                           ag_send.at[0], ag_recv.at[0], right)
            ag_to_l = copy(xbuf.at[c_own], xbuf.at[c_own],
                           ag_send.at[1], ag_recv.at[1], left)
            ag_to_d = copy(xbuf.at[c_own], xbuf.at[c_own],
                           ag_send.at[2], ag_recv.at[2], diag)
            ag_to_r.start()
            ag_to_l.start()
            ag_to_d.start()
            for s in rs_sends:
                s.wait_send()

            ag_fr_l = copy(xbuf.at[c_l], xbuf.at[c_l],
                           ag_send.at[0], ag_recv.at[0], left)
            ag_fr_r = copy(xbuf.at[c_r], xbuf.at[c_r],
                           ag_send.at[1], ag_recv.at[1], right)
            ag_fr_d = copy(xbuf.at[c_d], xbuf.at[c_d],
                           ag_send.at[2], ag_recv.at[2], diag)

            def rs_to(dev, slot, c):
                s = copy(part_ref.at[c], rcv_ref.at[slot],
                         rs_send.at[slot], rs_recv.at[slot], dev)
                s.start()
                return s

            if delta == 2:
                compute_chunk(c_own)
                s_d = rs_to(diag, 2, c_own)
                ag_fr_l.wait_recv()
                compute_chunk(c_l)
                s_r = rs_to(right, 0, c_l)
                ag_fr_r.wait_recv()
                compute_chunk(c_r)
                s_l = rs_to(left, 1, c_r)
                ag_fr_d.wait_recv()
                compute_chunk(c_d)
            else:
                compute_chunk(c_own)
                s_r = rs_to(right, 0, c_own)
                ag_fr_r.wait_recv()
                compute_chunk(c_r)
                s_d = rs_to(diag, 2, c_r)
                ag_fr_l.wait_recv()
                compute_chunk(c_l)
                ag_fr_d.wait_recv()
                compute_chunk(c_d)
                s_l = rs_to(left, 1, c_d)
            rs_sends = [s_l, s_r, s_d]

            ag_to_r.wait_send()
            ag_to_l.wait_send()
            ag_to_d.wait_send()

            rs_fr_l = copy(part_ref.at[c_keep], rcv_ref.at[0],
                           rs_send.at[0], rs_recv.at[0], left)
            rs_fr_r = copy(part_ref.at[c_keep], rcv_ref.at[1],
                           rs_send.at[1], rs_recv.at[1], right)
            rs_fr_d = copy(part_ref.at[c_keep], rcv_ref.at[2],
                           rs_send.at[2], rs_recv.at[2], diag)
            rs_fr_l.wait_recv()
            rs_fr_r.wait_recv()
            rs_fr_d.wait_recv()
            y = (part_ref[c_keep] + rcv_ref[0]) + (rcv_ref[1] + rcv_ref[2])
            if layer < 2:
                xnext[c_keep] = y
            else:
                out_ref[...] = y
                for s in rs_sends:
                    s.wait_send()

    return pl.pallas_call(
        body,
        out_shape=jax.ShapeDtypeStruct((m, d), jnp.float32),
        in_specs=[pl.BlockSpec(memory_space=pltpu.VMEM)]
        + [pl.BlockSpec(memory_space=pltpu.MemorySpace.HBM)] * 6,
        out_specs=pl.BlockSpec(memory_space=pltpu.VMEM),
        scratch_shapes=[
            pltpu.VMEM((N_DEV, m, d), jnp.float32),
            pltpu.VMEM((N_DEV, m, d), jnp.float32),
            pltpu.VMEM((N_DEV, m, d), jnp.float32),
            pltpu.VMEM((N_DEV - 1, m, d), jnp.float32),
            pltpu.VMEM((3,) + Win0.shape, jnp.float32),
            pltpu.VMEM((3,) + Wout0.shape, jnp.float32),
            pltpu.SemaphoreType.DMA((N_DEV - 1,)),
            pltpu.SemaphoreType.DMA((N_DEV - 1,)),
            pltpu.SemaphoreType.DMA((N_DEV - 1,)),
            pltpu.SemaphoreType.DMA((N_DEV - 1,)),
            pltpu.SemaphoreType.DMA((6,)),
        ],
        compiler_params=pltpu.CompilerParams(collective_id=0),
    )(x, Win0, Wout0, Win1, Wout1, Win2, Wout2)


# device time: 40565 ns/iter; 1.0194x vs baseline; 1.0136x over previous
import jax
import jax.numpy as jnp
from jax import lax
from jax.experimental import pallas as pl
from jax.experimental.pallas import tpu as pltpu

N_DEV = 4
R = [0, 2, 3, 0]


def kernel(x, Win0, Wout0, Win1, Wout1, Win2, Wout2):
    m, d = x.shape

    def body(x_ref, win0_ref, wout0_ref, win1_ref, wout1_ref,
             win2_ref, wout2_ref, out_ref,
             xbufA_ref, xbufB_ref, part_ref, rcv_ref,
             ag_send, ag_recv, rs_send, rs_recv):
        my = lax.axis_index("i")
        left = lax.rem(my + 3, N_DEV)
        right = lax.rem(my + 1, N_DEV)
        diag = lax.rem(my + 2, N_DEV)

        def cix(off):
            return lax.rem(my + (off % N_DEV), N_DEV)

        def copy(src, dst, ssem, rsem, dev):
            return pltpu.make_async_remote_copy(
                src_ref=src, dst_ref=dst, send_sem=ssem, recv_sem=rsem,
                device_id=(dev,), device_id_type=pl.DeviceIdType.MESH)

        barrier = pltpu.get_barrier_semaphore()
        for nbr in (left, right, diag):
            pl.semaphore_signal(barrier, inc=1, device_id=(nbr,),
                                device_id_type=pl.DeviceIdType.MESH)
        pl.semaphore_wait(barrier, 3)

        xbufA_ref[my] = x_ref[...]

        rs_sends = []

        wins = [win0_ref, win1_ref, win2_ref]
        wouts = [wout0_ref, wout1_ref, wout2_ref]

        for layer in range(3):
            win, wout = wins[layer], wouts[layer]
            xbuf = xbufA_ref if layer % 2 == 0 else xbufB_ref
            xnext = xbufB_ref if layer % 2 == 0 else xbufA_ref
            r, rp = R[layer], R[layer + 1]
            delta = (rp - r) % N_DEV
            c_own = cix(-r)
            c_l = cix(-1 - r)
            c_r = cix(1 - r)
            c_d = cix(2 - r)
            c_keep = cix(-rp)

            def compute_chunk(c):
                hc = jnp.maximum(
                    jnp.dot(xbuf[c], win[...],
                            preferred_element_type=jnp.float32), 0.0)
                part_ref[c] = jnp.dot(
                    hc, wout[...], preferred_element_type=jnp.float32)

            ag_to_r = copy(xbuf.at[c_own], xbuf.at[c_own],
                           ag_send.at[0], ag_recv.at[0], right)
            ag_to_l = copy(xbuf.at[c_own], xbuf.at[c_own],
                           ag_send.at[1], ag_recv.at[1], left)
            ag_to_d = copy(xbuf.at[c_own], xbuf.at[c_own],
                           ag_send.at[2], ag_recv.at[2], diag)
            ag_to_r.start()
            ag_to_l.start()
            ag_to_d.start()
            for s in rs_sends:
                s.wait_send()

            ag_fr_l = copy(xbuf.at[c_l], xbuf.at[c_l],
                           ag_send.at[0], ag_recv.at[0], left)
            ag_fr_r = copy(xbuf.at[c_r], xbuf.at[c_r],
                           ag_send.at[1], ag_recv.at[1], right)
            ag_fr_d = copy(xbuf.at[c_d], xbuf.at[c_d],
                           ag_send.at[2], ag_recv.at[2], diag)

            def rs_to(dev, slot, c):
                s = copy(part_ref.at[c], rcv_ref.at[slot],
                         rs_send.at[slot], rs_recv.at[slot], dev)
                s.start()
                return s

            if delta == 2:
                compute_chunk(c_own)
                s_d = rs_to(diag, 2, c_own)
                ag_fr_l.wait_recv()
                compute_chunk(c_l)
                s_r = rs_to(right, 0, c_l)
                ag_fr_r.wait_recv()
                compute_chunk(c_r)
                s_l = rs_to(left, 1, c_r)
                ag_fr_d.wait_recv()
                compute_chunk(c_d)
            else:
                compute_chunk(c_own)
                s_r = rs_to(right, 0, c_own)
                ag_fr_r.wait_recv()
                compute_chunk(c_r)
                s_d = rs_to(diag, 2, c_r)
                ag_fr_l.wait_recv()
                compute_chunk(c_l)
                ag_fr_d.wait_recv()
                compute_chunk(c_d)
                s_l = rs_to(left, 1, c_d)
            rs_sends = [s_l, s_r, s_d]

            ag_to_r.wait_send()
            ag_to_l.wait_send()
            ag_to_d.wait_send()

            rs_fr_l = copy(part_ref.at[c_keep], rcv_ref.at[0],
                           rs_send.at[0], rs_recv.at[0], left)
            rs_fr_r = copy(part_ref.at[c_keep], rcv_ref.at[1],
                           rs_send.at[1], rs_recv.at[1], right)
            rs_fr_d = copy(part_ref.at[c_keep], rcv_ref.at[2],
                           rs_send.at[2], rs_recv.at[2], diag)
            if delta == 2:
                rs_fr_d.wait_recv()
                y = part_ref[c_keep] + rcv_ref[2]
                rs_fr_l.wait_recv()
                y = y + rcv_ref[0]
                rs_fr_r.wait_recv()
                y = y + rcv_ref[1]
            else:
                rs_fr_l.wait_recv()
                y = part_ref[c_keep] + rcv_ref[0]
                rs_fr_d.wait_recv()
                y = y + rcv_ref[2]
                rs_fr_r.wait_recv()
                y = y + rcv_ref[1]
            if layer < 2:
                xnext[c_keep] = y
            else:
                out_ref[...] = y
                for s in rs_sends:
                    s.wait_send()

    return pl.pallas_call(
        body,
        out_shape=jax.ShapeDtypeStruct((m, d), jnp.float32),
        in_specs=[pl.BlockSpec(memory_space=pltpu.VMEM)] * 7,
        out_specs=pl.BlockSpec(memory_space=pltpu.VMEM),
        scratch_shapes=[
            pltpu.VMEM((N_DEV, m, d), jnp.float32),
            pltpu.VMEM((N_DEV, m, d), jnp.float32),
            pltpu.VMEM((N_DEV, m, d), jnp.float32),
            pltpu.VMEM((N_DEV - 1, m, d), jnp.float32),
            pltpu.SemaphoreType.DMA((N_DEV - 1,)),
            pltpu.SemaphoreType.DMA((N_DEV - 1,)),
            pltpu.SemaphoreType.DMA((N_DEV - 1,)),
            pltpu.SemaphoreType.DMA((N_DEV - 1,)),
        ],
        compiler_params=pltpu.CompilerParams(collective_id=0),
    )(x, Win0, Wout0, Win1, Wout1, Win2, Wout2)
